# Initial kernel scaffold; baseline (speedup 1.0000x reference)
import jax
import jax.numpy as jnp
from jax.experimental import pallas as pl


def kernel(A_pad, edges, node_idx_gnn, node_idx_feat, att_idx, params):
    raise NotImplementedError("write your pallas kernel here")


# speedup vs baseline: 2.2192x; 2.2192x over previous
import functools

import jax
import jax.numpy as jnp
from jax import lax
from jax.experimental import pallas as pl
from jax.experimental.pallas import tpu as pltpu
from jax.experimental.pallas import tpu_sc as plsc

_HID = 128
_ATT = 64
_K = 20
_NC = 2
_NS = 16
_NW = _NC * _NS


def _sc_mesh():
    return plsc.VectorSubcoreMesh(core_axis_name="c", subcore_axis_name="s")


_SC_PARAMS = pltpu.CompilerParams(needs_layout_passes=False)


def _widx():
    return lax.axis_index("s") * _NC + lax.axis_index("c")


def _gather_rows_sc(table, idx):
    B = idx.shape[0]
    D = table.shape[1]
    per = B // _NW
    ch = 64
    nch = per // ch

    @functools.partial(
        pl.kernel,
        mesh=_sc_mesh(),
        out_type=jax.ShapeDtypeStruct((B, D), jnp.float32),
        compiler_params=_SC_PARAMS,
        scratch_types=[
            pltpu.VMEM((ch,), jnp.int32),
            pltpu.VMEM((ch, D), jnp.float32),
            pltpu.SemaphoreType.DMA,
        ],
    )
    def k(tab_h, idx_h, out_h, idx_v, rows_v, sem):
        base = _widx() * per

        def chunk(i, c):
            off = base + i * ch
            pltpu.sync_copy(idx_h.at[pl.ds(off, ch)], idx_v)
            pltpu.async_copy(tab_h.at[idx_v], rows_v, sem).wait()
            pltpu.sync_copy(rows_v, out_h.at[pl.ds(off, ch)])
            return c

        lax.fori_loop(0, nch, chunk, 0)

    return k(table, idx)


def _gather_pairs_sc(table, i0, i1, att=None):
    B = i0.shape[0]
    D = table.shape[1]
    per = B // _NW
    ch = 128
    nch = per // ch
    with_att = att is not None

    out_type = [
        jax.ShapeDtypeStruct((B, D), jnp.float32),
        jax.ShapeDtypeStruct((B, D), jnp.float32),
    ]
    scratch = [
        pltpu.VMEM((ch,), jnp.int32),
        pltpu.VMEM((ch,), jnp.int32),
        pltpu.VMEM((ch, D), jnp.float32),
        pltpu.VMEM((ch, D), jnp.float32),
        pltpu.SemaphoreType.DMA,
        pltpu.SemaphoreType.DMA,
    ]
    if with_att:
        out_type += [
            jax.ShapeDtypeStruct((B,), jnp.int32),
            jax.ShapeDtypeStruct((B,), jnp.int32),
        ]
        scratch += [
            pltpu.VMEM((att.shape[0],), jnp.int32),
            pltpu.VMEM((ch,), jnp.int32),
            pltpu.VMEM((ch,), jnp.int32),
        ]

    def body(tab_h, i0_h, i1_h, *rest):
        if with_att:
            (att_h, r0_h, r1_h, a0_h, a1_h,
             idx0, idx1, rows0, rows1, sem0, sem1, att_v, a0_v, a1_v) = rest
        else:
            (r0_h, r1_h, idx0, idx1, rows0, rows1, sem0, sem1) = rest
        base = _widx() * per
        if with_att:
            pltpu.sync_copy(att_h, att_v)

        def chunk(i, c):
            off = base + i * ch
            pltpu.sync_copy(i0_h.at[pl.ds(off, ch)], idx0)
            pltpu.sync_copy(i1_h.at[pl.ds(off, ch)], idx1)
            cp0 = pltpu.async_copy(tab_h.at[idx0], rows0, sem0)
            cp1 = pltpu.async_copy(tab_h.at[idx1], rows1, sem1)
            if with_att:
                for j in range(ch // 16):
                    s = pl.ds(j * 16, 16)
                    a0_v[s] = plsc.load_gather(att_v, [idx0[s]])
                    a1_v[s] = plsc.load_gather(att_v, [idx1[s]])
            cp0.wait()
            pltpu.sync_copy(rows0, r0_h.at[pl.ds(off, ch)])
            cp1.wait()
            pltpu.sync_copy(rows1, r1_h.at[pl.ds(off, ch)])
            if with_att:
                pltpu.sync_copy(a0_v, a0_h.at[pl.ds(off, ch)])
                pltpu.sync_copy(a1_v, a1_h.at[pl.ds(off, ch)])
            return c

        lax.fori_loop(0, nch, chunk, 0)

    args = (table, i0, i1) + ((att,) if with_att else ())
    return pl.kernel(
        body, mesh=_sc_mesh(), out_type=out_type, scratch_types=scratch,
        compiler_params=_SC_PARAMS,
    )(*args)


def _scatter_add_sc(msg, dst, n_rows):
    B = msg.shape[0]
    D = msg.shape[1]
    half = B // 2
    per = half // _NS
    ch = 128
    nch = per // ch
    zr = 64
    rows_per_tile = n_rows // _NS
    nzb = rows_per_tile // zr

    @functools.partial(
        pl.kernel,
        mesh=_sc_mesh(),
        out_type=jax.ShapeDtypeStruct((2 * n_rows, D), jnp.float32),
        compiler_params=_SC_PARAMS,
        scratch_types=[
            pltpu.VMEM_SHARED((n_rows, D), jnp.float32),
            pltpu.VMEM((ch,), jnp.int32),
            pltpu.VMEM((ch, D), jnp.float32),
            pltpu.VMEM((zr, D), jnp.float32),
        ],
    )
    def k(msg_h, dst_h, out_h, acc_sh, idx_v, rows_v, zero_v):
        c = lax.axis_index("c")
        s = lax.axis_index("s")

        def zrow(r, carry):
            for j in range(D // 16):
                zero_v[r, pl.ds(j * 16, 16)] = jnp.zeros((16,), jnp.float32)
            return carry

        lax.fori_loop(0, zr, zrow, 0)

        def zcp(t, carry):
            pltpu.sync_copy(zero_v, acc_sh.at[pl.ds(s * rows_per_tile + t * zr, zr)])
            return carry

        lax.fori_loop(0, nzb, zcp, 0)
        plsc.subcore_barrier()

        base = c * half + s * per

        def chunk(i, carry):
            off = base + i * ch
            pltpu.sync_copy(dst_h.at[pl.ds(off, ch)], idx_v)
            pltpu.sync_copy(msg_h.at[pl.ds(off, ch)], rows_v)
            pltpu.sync_copy(rows_v, acc_sh.at[idx_v], add=True)
            return carry

        lax.fori_loop(0, nch, chunk, 0)
        plsc.subcore_barrier()

        def wb(t, carry):
            r0 = s * rows_per_tile + t * zr
            pltpu.sync_copy(acc_sh.at[pl.ds(r0, zr)],
                            out_h.at[pl.ds(c * n_rows + r0, zr)])
            return carry

        lax.fori_loop(0, nzb, wb, 0)

    return k(msg, dst)


def _mm_t(x, w):
    return lax.dot_general(x, w, (((1,), (1,)), ((), ())),
                           preferred_element_type=jnp.float32)


def _node_feat_tc(A, W, b):
    M, Kin = A.shape
    blk = 2000
    grid = (M // blk,)

    def body(a_r, w_r, b_r, o_r):
        o_r[...] = _mm_t(a_r[...], w_r[...]) + b_r[...]

    return pl.pallas_call(
        body,
        grid=grid,
        in_specs=[
            pl.BlockSpec((blk, Kin), lambda i: (i, 0)),
            pl.BlockSpec(W.shape, lambda i: (0, 0)),
            pl.BlockSpec(b.shape, lambda i: (0, 0)),
        ],
        out_specs=pl.BlockSpec((blk, _HID), lambda i: (i, 0)),
        out_shape=jax.ShapeDtypeStruct((M, _HID), jnp.float32),
    )(A, W, b)


def _edge_mlp_tc(r0, r1, a0, a1, w1s, w1a, b1, wm2, bm2, wa2, ba2):
    B = r0.shape[0]
    blk = 2048
    grid = (B // blk,)

    def body(r0_r, r1_r, a0_r, a1_r, w1s_r, w1a_r, b1_r, wm2_r, bm2_r,
             wa2_r, ba2_r, o_r):
        diff = r0_r[...] - r1_r[...]
        iot = lax.broadcasted_iota(jnp.int32, (blk, 2 * _ATT), 1)
        sel = jnp.where(iot < _ATT, a0_r[...], a1_r[...] + _ATT)
        oh = (sel == iot).astype(jnp.float32)
        pre = _mm_t(diff, w1s_r[...]) + _mm_t(oh, w1a_r[...]) + b1_r[...]
        h = jnp.maximum(pre, 0.0)
        msg0 = _mm_t(h[:, :_HID], wm2_r[...]) + bm2_r[...]
        attw = jax.nn.sigmoid(_mm_t(h[:, _HID:], wa2_r[...]) + ba2_r[...])
        o_r[...] = msg0 * attw

    full = lambda a: pl.BlockSpec(a.shape, lambda i: (0,) * a.ndim)
    return pl.pallas_call(
        body,
        grid=grid,
        in_specs=[
            pl.BlockSpec((blk, _HID), lambda i: (i, 0)),
            pl.BlockSpec((blk, _HID), lambda i: (i, 0)),
            pl.BlockSpec((blk, 1), lambda i: (i, 0)),
            pl.BlockSpec((blk, 1), lambda i: (i, 0)),
            full(w1s), full(w1a), full(b1), full(wm2), full(bm2),
            full(wa2), full(ba2),
        ],
        out_specs=pl.BlockSpec((blk, _HID), lambda i: (i, 0)),
        out_shape=jax.ShapeDtypeStruct((B, _HID), jnp.float32),
    )(r0, r1, a0, a1, w1s, w1a, b1, wm2, bm2, wa2, ba2)


def _gru_tc(p0, p1, st, wih, whh, bih, bhh):
    M = st.shape[0]
    blk = 2048
    grid = (M // blk,)

    def body(p0_r, p1_r, st_r, wih_r, whh_r, bih_r, bhh_r, o_r):
        sm = p0_r[...] + p1_r[...]
        s = st_r[...]
        gi = _mm_t(sm, wih_r[...]) + bih_r[...]
        gh = _mm_t(s, whh_r[...]) + bhh_r[...]
        r = jax.nn.sigmoid(gi[:, :_HID] + gh[:, :_HID])
        z = jax.nn.sigmoid(gi[:, _HID:2 * _HID] + gh[:, _HID:2 * _HID])
        n = jnp.tanh(gi[:, 2 * _HID:] + r * gh[:, 2 * _HID:])
        o_r[...] = (1.0 - z) * n + z * s

    full = lambda a: pl.BlockSpec(a.shape, lambda i: (0,) * a.ndim)
    return pl.pallas_call(
        body,
        grid=grid,
        in_specs=[
            pl.BlockSpec((blk, _HID), lambda i: (i, 0)),
            pl.BlockSpec((blk, _HID), lambda i: (i, 0)),
            pl.BlockSpec((blk, _HID), lambda i: (i, 0)),
            full(wih), full(whh), full(bih), full(bhh),
        ],
        out_specs=pl.BlockSpec((blk, _HID), lambda i: (i, 0)),
        out_shape=jax.ShapeDtypeStruct((M, _HID), jnp.float32),
    )(p0, p1, st, wih, whh, bih, bhh)


def _heads_tc(d0, d1, w1, b1, wt2, bt2, wp2, bp2, wt3, bt3, wp3, bp3):
    B = d0.shape[0]
    blk = 2048
    grid = (B // blk,)

    def body(d0_r, d1_r, w1_r, b1_r, wt2_r, bt2_r, wp2_r, bp2_r, wt3_r,
             bt3_r, wp3_r, bp3_r, ot_r, oa_r):
        diff = d0_r[...] - d1_r[...]
        h = jnp.maximum(_mm_t(diff, w1_r[...]) + b1_r[...], 0.0)
        t = jnp.maximum(_mm_t(h[:, :_HID], wt2_r[...]) + bt2_r[...], 0.0)
        ot_r[...] = _mm_t(t, wt3_r[...]) + bt3_r[...]
        a = jnp.maximum(_mm_t(h[:, _HID:], wp2_r[...]) + bp2_r[...], 0.0)
        lp = _mm_t(a, wp3_r[...]) + bp3_r[...]
        m = jnp.max(lp, axis=1, keepdims=True)
        ls = jnp.log(jnp.sum(jnp.exp(lp - m), axis=1, keepdims=True)) + m
        oa_r[...] = lp - ls

    full = lambda a: pl.BlockSpec(a.shape, lambda i: (0,) * a.ndim)
    return pl.pallas_call(
        body,
        grid=grid,
        in_specs=[
            pl.BlockSpec((blk, _HID), lambda i: (i, 0)),
            pl.BlockSpec((blk, _HID), lambda i: (i, 0)),
            full(w1), full(b1), full(wt2), full(bt2), full(wp2), full(bp2),
            full(wt3), full(bt3), full(wp3), full(bp3),
        ],
        out_specs=[
            pl.BlockSpec((blk, _K), lambda i: (i, 0)),
            pl.BlockSpec((blk, _K), lambda i: (i, 0)),
        ],
        out_shape=[
            jax.ShapeDtypeStruct((B, _K), jnp.float32),
            jax.ShapeDtypeStruct((B, _K), jnp.float32),
        ],
    )(d0, d1, w1, b1, wt2, bt2, wp2, bp2, wt3, bt3, wp3, bp3)


def kernel(A_pad, edges, node_idx_gnn, node_idx_feat, att_idx, params):
    p = params
    Bb, Cc, Nm, _ = A_pad.shape
    A = A_pad.reshape(Bb * Cc * Nm, Nm)
    NN = A.shape[0]
    NE = edges.shape[0]
    EPAD = -(-NE // 4096) * 4096
    NPAD = -(-NN // 2048) * 2048

    nf = _node_feat_tc(A, p['W_di'], p['b_di'].reshape(1, -1))
    nf_pad = jnp.pad(nf, ((1, 0), (0, 0)))
    nif = jnp.concatenate(
        [node_idx_feat.astype(jnp.int32),
         jnp.zeros((NPAD - NN,), jnp.int32)])
    state = _gather_rows_sc(nf_pad, nif)

    epad0 = jnp.zeros((EPAD - NE,), jnp.int32)
    epad1 = jnp.full((EPAD - NE,), NPAD - 1, jnp.int32)
    e0 = jnp.concatenate([edges[:, 0].astype(jnp.int32), epad0])
    e1 = jnp.concatenate([edges[:, 1].astype(jnp.int32), epad1])
    r0, r1, a0, a1 = _gather_pairs_sc(state, e0, e1, att_idx.astype(jnp.int32))

    w1s = jnp.concatenate([p['W_m1'][:, :_HID], p['W_a1'][:, :_HID]], axis=0)
    w1a = jnp.concatenate([p['W_m1'][:, _HID:], p['W_a1'][:, _HID:]], axis=0)
    b1e = jnp.concatenate([p['b_m1'], p['b_a1']]).reshape(1, -1)
    msg = _edge_mlp_tc(r0, r1, a0.reshape(-1, 1), a1.reshape(-1, 1),
                       w1s, w1a, b1e,
                       p['W_m2'], p['b_m2'].reshape(1, -1),
                       p['W_a2'], p['b_a2'].reshape(1, -1))

    parts = _scatter_add_sc(msg, e1, NPAD)

    new_state = _gru_tc(parts[:NPAD], parts[NPAD:], state,
                        p['W_ih'], p['W_hh'],
                        p['b_ih'].reshape(1, -1), p['b_hh'].reshape(1, -1))

    n0 = jnp.concatenate([node_idx_gnn[:, 0].astype(jnp.int32), epad0])
    n1 = jnp.concatenate([node_idx_gnn[:, 1].astype(jnp.int32), epad0])
    d0, d1 = _gather_pairs_sc(new_state, n0, n1)

    w1h = jnp.concatenate([p['W_t1'], p['W_p1']], axis=0)
    b1h = jnp.concatenate([p['b_t1'], p['b_p1']]).reshape(1, -1)
    log_theta, log_alpha = _heads_tc(
        d0, d1, w1h, b1h,
        p['W_t2'], p['b_t2'].reshape(1, -1),
        p['W_p2'], p['b_p2'].reshape(1, -1),
        p['W_t3'], p['b_t3'].reshape(1, -1),
        p['W_p3'], p['b_p3'].reshape(1, -1))
    return log_theta[:NE], log_alpha[:NE]
